# baseline (speedup 1.0000x reference)
import functools

import jax
import jax.numpy as jnp
from jax import lax
from jax.experimental import pallas as pl
from jax.experimental.pallas import tpu as pltpu
from jax.experimental.pallas import tpu_sc as plsc

_B, _D, _T = 16, 768, 576
_K, _CD = 8192, 64
_N = _B * _T

_NC, _NS = 2, 16
_NW = _NC * _NS


def _make_assign_body(emit_cbp):
    def body(x_ref, w_in_ref, b_in_ref, cb_ref, codes_ref, loss_ref, *rest):
        b = pl.program_id(0)
        if emit_cbp:
            cbp_ref = rest[0]

            @pl.when(b == 0)
            def _():
                cbp_ref[:, :_CD] = cb_ref[...]
                cbp_ref[:, _CD:] = jnp.zeros((_K, 128 - _CD), jnp.float32)

        xb = x_ref[0]
        z = lax.dot_general(xb, w_in_ref[...], (((0,), (0,)), ((), ())),
                            preferred_element_type=jnp.float32)
        z = z + b_in_ref[...]
        zn = jnp.sum(z * z, axis=1, keepdims=True)
        n_win = 4
        wk = _K // n_win
        lane_iota = lax.broadcasted_iota(jnp.int32, (_T, 128), 1)
        rv = ri = sel = None
        for w in range(n_win):
            chunk = cb_ref[w * wk:(w + 1) * wk, :]
            cn = jnp.sum(chunk * chunk, axis=1)[None, :]
            t2n = lax.dot_general(z, -2.0 * chunk, (((1,), (1,)), ((), ())),
                                  preferred_element_type=jnp.float32)
            rm = ridx = None
            for cb_i in range(wk // 128):
                sb = (zn + t2n[:, cb_i * 128:(cb_i + 1) * 128]) \
                    + cn[:, cb_i * 128:(cb_i + 1) * 128]
                ib = lane_iota + (w * wk + cb_i * 128)
                if cb_i == 0:
                    rm, ridx = sb, ib
                else:
                    lt = sb < rm
                    rm = jnp.minimum(rm, sb)
                    ridx = jnp.where(lt, ib, ridx)
            gmin = jnp.min(rm, axis=1, keepdims=True)
            gidx = jnp.min(jnp.where(rm == gmin, ridx, _K),
                           axis=1, keepdims=True)
            gmin_b = gmin.astype(jnp.bfloat16).astype(jnp.float32)
            if w == 0:
                rv, ri, sel = gmin_b, gidx, gmin
            else:
                upd = gmin < rv
                ri = jnp.where(upd, gidx, ri)
                sel = jnp.where(upd, gmin, sel)
                rv = jnp.where(upd, gmin_b, rv)
        codes_ref[0, 0, :] = ri[:, 0]
        partial = jnp.sum(sel)

        @pl.when(b == 0)
        def _():
            loss_ref[0, 0] = partial

        @pl.when(b > 0)
        def _():
            loss_ref[0, 0] = loss_ref[0, 0] + partial

    return body


def _assign(x, w_in, b_in, codebook, emit_cbp):
    nb = x.shape[0]
    out_specs = [
        pl.BlockSpec((1, 1, _T), lambda b: (b, 0, 0)),
        pl.BlockSpec(memory_space=pltpu.SMEM),
    ]
    out_shape = [
        jax.ShapeDtypeStruct((nb, 1, _T), jnp.int32),
        jax.ShapeDtypeStruct((1, 1), jnp.float32),
    ]
    if emit_cbp:
        out_specs.append(pl.BlockSpec((_K, 128), lambda b: (0, 0)))
        out_shape.append(jax.ShapeDtypeStruct((_K, 128), jnp.float32))
    return pl.pallas_call(
        _make_assign_body(emit_cbp),
        grid=(nb,),
        in_specs=[
            pl.BlockSpec((1, _D, _T), lambda b: (b, 0, 0)),
            pl.BlockSpec((_D, _CD), lambda b: (0, 0)),
            pl.BlockSpec((1, _CD), lambda b: (0, 0)),
            pl.BlockSpec((_K, _CD), lambda b: (0, 0)),
        ],
        out_specs=out_specs,
        out_shape=out_shape,
    )(x, w_in, b_in.reshape(1, _CD), codebook)


def _gather(cbp, codes_flat):
    n = codes_flat.shape[0]
    rows_w = n // _NW
    gch = 72 if rows_w % 72 == 0 else 96
    ngch = rows_w // gch
    mesh = plsc.VectorSubcoreMesh(core_axis_name="c", subcore_axis_name="s")

    @functools.partial(
        pl.kernel,
        mesh=mesh,
        out_type=jax.ShapeDtypeStruct((_NW, ngch, gch, 128), jnp.float32),
        scratch_types=[
            pltpu.VMEM((ngch, gch), jnp.int32),
            pltpu.VMEM((ngch, gch, 128), jnp.float32),
            pltpu.SemaphoreType.DMA,
        ],
    )
    def gather_k(cb_hbm, idx_hbm, out_hbm, idx_v, rows_v, sem):
        wid = lax.axis_index("s") * _NC + lax.axis_index("c")
        pltpu.sync_copy(idx_hbm.at[wid], idx_v)
        copies = [pltpu.async_copy(cb_hbm.at[idx_v.at[j]], rows_v.at[j], sem)
                  for j in range(ngch)]
        for c in copies:
            c.wait()
        pltpu.sync_copy(rows_v, out_hbm.at[wid])

    idx = codes_flat.reshape(_NW, ngch, gch)
    out = gather_k(cbp, idx)
    return out.reshape(n, 128)


def _proj_body(q_ref, w_out_ref, b_out_ref, out_ref):
    qb = q_ref[0][:, :_CD]
    ob = lax.dot_general(w_out_ref[...], qb, (((0,), (1,)), ((), ())),
                         preferred_element_type=jnp.float32)
    out_ref[0] = ob + b_out_ref[...]


def _proj(q, w_out, b_out):
    nb = q.shape[0]
    return pl.pallas_call(
        _proj_body,
        grid=(nb,),
        in_specs=[
            pl.BlockSpec((1, _T, 128), lambda b: (b, 0, 0)),
            pl.BlockSpec((_CD, _D), lambda b: (0, 0)),
            pl.BlockSpec((_D, 1), lambda b: (0, 0)),
        ],
        out_specs=pl.BlockSpec((1, _D, _T), lambda b: (b, 0, 0)),
        out_shape=jax.ShapeDtypeStruct((nb, _D, _T), jnp.float32),
    )(q, w_out, b_out.reshape(_D, 1))


def kernel(x, w_in, b_in, w_out, b_out, codebook):
    h = _B // 2
    codes0, loss0, cbp = _assign(x[:h], w_in, b_in, codebook, True)
    q0 = _gather(cbp, codes0.reshape(h * _T))
    codes1, loss1 = _assign(x[h:], w_in, b_in, codebook, False)
    q1 = _gather(cbp, codes1.reshape(h * _T))
    q = jnp.concatenate([q0.reshape(h, _T, 128), q1.reshape(h, _T, 128)],
                        axis=0)
    out = _proj(q, w_out, b_out)
    codes_btc = jnp.concatenate([codes0.reshape(h, _T),
                                 codes1.reshape(h, _T)], axis=0)
    commit_loss = (loss0[0, 0] + loss1[0, 0]) / jnp.float32(_N * _CD)
    return out, codes_btc, commit_loss


# speedup vs baseline: 1.0927x; 1.0927x over previous
import functools

import jax
import jax.numpy as jnp
from jax import lax
from jax.experimental import pallas as pl
from jax.experimental.pallas import tpu as pltpu
from jax.experimental.pallas import tpu_sc as plsc

_B, _D, _T = 16, 768, 576
_K, _CD = 8192, 64
_KC = 1024
_NKC = _K // _KC
_N = _B * _T

_NC, _NS = 2, 16
_NW = _NC * _NS
_ROWS_W = _N // _NW
_GCH = 96
_NGCH = _ROWS_W // _GCH


def _assign_body(x_ref, w_in_ref, b_in_ref, cb_ref, codes_ref, loss_ref,
                 cbp_ref):
    b = pl.program_id(0)

    @pl.when(b == 0)
    def _():
        cbp_ref[:, :_CD] = cb_ref[...]
        cbp_ref[:, _CD:] = jnp.zeros((_K, 128 - _CD), jnp.float32)

    xb = x_ref[0]
    z = lax.dot_general(xb, w_in_ref[...], (((0,), (0,)), ((), ())),
                        preferred_element_type=jnp.float32)
    z = z + b_in_ref[...]
    zn = jnp.sum(z * z, axis=1, keepdims=True)
    n_win = 4
    wk = _K // n_win
    lane_iota = lax.broadcasted_iota(jnp.int32, (_T, 128), 1)
    rv = ri = sel = None
    for w in range(n_win):
        chunk = cb_ref[w * wk:(w + 1) * wk, :]
        cn = jnp.sum(chunk * chunk, axis=1)[None, :]
        t2n = lax.dot_general(z, -2.0 * chunk, (((1,), (1,)), ((), ())),
                              preferred_element_type=jnp.float32)
        rm = ridx = None
        for cb_i in range(wk // 128):
            sb = (zn + t2n[:, cb_i * 128:(cb_i + 1) * 128]) \
                + cn[:, cb_i * 128:(cb_i + 1) * 128]
            ib = lane_iota + (w * wk + cb_i * 128)
            if cb_i == 0:
                rm, ridx = sb, ib
            else:
                lt = sb < rm
                rm = jnp.minimum(rm, sb)
                ridx = jnp.where(lt, ib, ridx)
        gmin = jnp.min(rm, axis=1, keepdims=True)
        gidx = jnp.min(jnp.where(rm == gmin, ridx, _K), axis=1, keepdims=True)
        gmin_b = gmin.astype(jnp.bfloat16).astype(jnp.float32)
        if w == 0:
            rv, ri, sel = gmin_b, gidx, gmin
        else:
            upd = gmin < rv
            ri = jnp.where(upd, gidx, ri)
            sel = jnp.where(upd, gmin, sel)
            rv = jnp.where(upd, gmin_b, rv)
    codes_ref[0, 0, :] = ri[:, 0]
    partial = jnp.sum(sel)

    @pl.when(b == 0)
    def _():
        loss_ref[0, 0] = partial

    @pl.when(b > 0)
    def _():
        loss_ref[0, 0] = loss_ref[0, 0] + partial


def _assign(x, w_in, b_in, codebook):
    return pl.pallas_call(
        _assign_body,
        grid=(_B,),
        in_specs=[
            pl.BlockSpec((1, _D, _T), lambda b: (b, 0, 0)),
            pl.BlockSpec((_D, _CD), lambda b: (0, 0)),
            pl.BlockSpec((1, _CD), lambda b: (0, 0)),
            pl.BlockSpec((_K, _CD), lambda b: (0, 0)),
        ],
        out_specs=[
            pl.BlockSpec((1, 1, _T), lambda b: (b, 0, 0)),
            pl.BlockSpec(memory_space=pltpu.SMEM),
            pl.BlockSpec((_K, 128), lambda b: (0, 0)),
        ],
        out_shape=[
            jax.ShapeDtypeStruct((_B, 1, _T), jnp.int32),
            jax.ShapeDtypeStruct((1, 1), jnp.float32),
            jax.ShapeDtypeStruct((_K, 128), jnp.float32),
        ],
    )(x, w_in, b_in.reshape(1, _CD), codebook)


def _gather(codebook, codes_flat):
    mesh = plsc.VectorSubcoreMesh(core_axis_name="c", subcore_axis_name="s")

    @functools.partial(
        pl.kernel,
        mesh=mesh,
        out_type=jax.ShapeDtypeStruct((_NW, _NGCH, _GCH, 128), jnp.float32),
        scratch_types=[
            pltpu.VMEM((_NGCH, _GCH), jnp.int32),
            pltpu.VMEM((_NGCH, _GCH, 128), jnp.float32),
            pltpu.SemaphoreType.DMA,
        ],
    )
    def gather_k(cb_hbm, idx_hbm, out_hbm, idx_v, rows_v, sem):
        wid = lax.axis_index("s") * _NC + lax.axis_index("c")
        pltpu.sync_copy(idx_hbm.at[wid], idx_v)
        copies = [pltpu.async_copy(cb_hbm.at[idx_v.at[j]], rows_v.at[j], sem)
                  for j in range(_NGCH)]
        for c in copies:
            c.wait()
        pltpu.sync_copy(rows_v, out_hbm.at[wid])

    idx = codes_flat.reshape(_NW, _NGCH, _GCH)
    out = gather_k(codebook, idx)
    return out.reshape(_N, 128)


def _proj_body(q_ref, w_out_ref, b_out_ref, out_ref):
    qb = q_ref[0][:, :_CD]
    ob = lax.dot_general(w_out_ref[...], qb, (((0,), (1,)), ((), ())),
                         preferred_element_type=jnp.float32)
    out_ref[0] = ob + b_out_ref[...]


def _proj(q, w_out, b_out):
    return pl.pallas_call(
        _proj_body,
        grid=(_B,),
        in_specs=[
            pl.BlockSpec((1, _T, 128), lambda b: (b, 0, 0)),
            pl.BlockSpec((_CD, _D), lambda b: (0, 0)),
            pl.BlockSpec((_D, 1), lambda b: (0, 0)),
        ],
        out_specs=pl.BlockSpec((1, _D, _T), lambda b: (b, 0, 0)),
        out_shape=jax.ShapeDtypeStruct((_B, _D, _T), jnp.float32),
    )(q, w_out, b_out.reshape(_D, 1))


def kernel(x, w_in, b_in, w_out, b_out, codebook):
    codes3, loss11, cbp = _assign(x, w_in, b_in, codebook)
    codes_flat = codes3.reshape(_N)
    q = _gather(cbp, codes_flat)
    out = _proj(q.reshape(_B, _T, 128), w_out, b_out)
    codes_btc = codes3.reshape(_B, _T)
    commit_loss = loss11[0, 0] / jnp.float32(_N * _CD)
    return out, codes_btc, commit_loss
